# Initial kernel scaffold; baseline (speedup 1.0000x reference)
import jax
import jax.numpy as jnp
from jax.experimental import pallas as pl


def kernel(x):
    raise NotImplementedError("write your pallas kernel here")


# speedup vs baseline: 2.1353x; 2.1353x over previous
import functools

import jax
import jax.numpy as jnp
from jax import lax
from jax.experimental import pallas as pl
from jax.experimental.pallas import tpu as pltpu
from jax.experimental.pallas import tpu_sc as plsc

B, N = 64, 32768
NC, NS, L = 2, 16, 16
NW = NC * NS
ROWS_PER_W = B // NW
NV = N // L
BIS_ITERS = 30
NEG = -1e30

_mesh = plsc.VectorSubcoreMesh(core_axis_name="c", subcore_axis_name="s")


@functools.partial(
    pl.kernel,
    out_type=jax.ShapeDtypeStruct((B, N), jnp.float32),
    mesh=_mesh,
    compiler_params=pltpu.CompilerParams(needs_layout_passes=False),
    scratch_types=[
        pltpu.VMEM((N,), jnp.float32),
        pltpu.VMEM((N,), jnp.float32),
    ],
)
def _entmax_sc(x_hbm, out_hbm, row_v, y_v):
    wid = lax.axis_index("s") * NC + lax.axis_index("c")

    for r in range(ROWS_PER_W):
        row_id = wid * ROWS_PER_W + r
        pltpu.sync_copy(x_hbm.at[row_id], row_v)

        def body_max(i, acc):
            return jnp.maximum(acc, row_v[pl.ds(i * L, L)])

        m = jnp.max(lax.fori_loop(0, NV, body_max, jnp.full((L,), NEG, jnp.float32)))

        def bis(_, carry):
            lo, hi = carry
            mid = 0.5 * (lo + hi)

            def body_g(i, acc):
                return acc + jnp.maximum(row_v[pl.ds(i * L, L)] - mid, 0.0)

            g = jnp.sum(lax.fori_loop(0, NV, body_g, jnp.zeros((L,), jnp.float32)))
            pred = g >= 0.5
            return jnp.where(pred, mid, lo), jnp.where(pred, hi, mid)

        t, _ = lax.fori_loop(0, BIS_ITERS, bis, (m - 1.0, m))

        def body_stats(i, carry):
            cntv, sv, bv = carry
            v = row_v[pl.ds(i * L, L)]
            z = v - m
            msk = v > t
            cntv = cntv + jnp.where(msk, 1.0, 0.0)
            sv = sv + jnp.where(msk, z, 0.0)
            bv = jnp.maximum(bv, jnp.where(msk, NEG, z))
            return cntv, sv, bv

        cntv, sv, bv = lax.fori_loop(
            0,
            NV,
            body_stats,
            (
                jnp.zeros((L,), jnp.float32),
                jnp.zeros((L,), jnp.float32),
                jnp.full((L,), NEG, jnp.float32),
            ),
        )
        cnt = jnp.sum(cntv)
        s_above = jnp.sum(sv)
        v_next = jnp.max(bv)
        num = jnp.full((L,), s_above + v_next - 0.5, jnp.float32)
        den = jnp.full((L,), cnt + 1.0, jnp.float32)
        tau_abs = jnp.full((L,), v_next + m, jnp.float32) - num / den

        def body_y(i, acc):
            z = row_v[pl.ds(i * L, L)] - tau_abs
            pos = z > 0.0
            zi = plsc.bitcast(z, jnp.int32)
            s0 = plsc.bitcast((zi >> 1) + 0x1FBD1DF6, jnp.float32)
            s1 = 0.5 * (s0 + z / s0)
            s2 = 0.5 * (s1 + z / s1)
            s3 = 0.5 * (s2 + z / s2)
            y = jnp.where(pos, s3, 0.0)
            y_v[pl.ds(i * L, L)] = y
            return acc + y

        yacc = lax.fori_loop(0, NV, body_y, jnp.zeros((L,), jnp.float32))
        inv = jnp.ones((L,), jnp.float32) / jnp.full((L,), jnp.sum(yacc), jnp.float32)

        def body_scale(i, _):
            y_v[pl.ds(i * L, L)] = y_v[pl.ds(i * L, L)] * inv
            return 0

        lax.fori_loop(0, NV, body_scale, 0)
        pltpu.sync_copy(y_v, out_hbm.at[row_id])


def kernel(x):
    return _entmax_sc(x)


# speedup vs baseline: 25.8965x; 12.1277x over previous
import functools

import jax
import jax.numpy as jnp
from jax import lax
from jax.experimental import pallas as pl
from jax.experimental.pallas import tpu as pltpu
from jax.experimental.pallas import tpu_sc as plsc

B, N = 64, 32768
NC, NS, L = 2, 16, 16
NW = NC * NS
ROWS_PER_W = B // NW
NV = N // L
CV = 8
CH = CV * L
NCH = N // CH
CAP = 8192
BIS_ITERS = 30
NEG = -1e30

_mesh = plsc.VectorSubcoreMesh(core_axis_name="c", subcore_axis_name="s")


def _sqrt16(z):
    zi = plsc.bitcast(z, jnp.int32)
    s0 = plsc.bitcast((zi >> 1) + 0x1FBD1DF6, jnp.float32)
    s1 = 0.5 * (s0 + z / s0)
    s2 = 0.5 * (s1 + z / s1)
    return 0.5 * (s2 + z / s2)


@functools.partial(
    pl.kernel,
    out_type=jax.ShapeDtypeStruct((B, N), jnp.float32),
    mesh=_mesh,
    compiler_params=pltpu.CompilerParams(needs_layout_passes=False),
    scratch_types=[
        pltpu.VMEM((N,), jnp.float32),
        pltpu.VMEM((N,), jnp.float32),
        pltpu.VMEM((NCH * L,), jnp.float32),
        pltpu.VMEM((CAP + L,), jnp.float32),
        pltpu.VMEM((CAP + L,), jnp.int32),
    ],
)
def _entmax_sc(x_hbm, out_hbm, row_v, y_v, cm_v, cand_v, cidx_v):
    wid = lax.axis_index("s") * NC + lax.axis_index("c")
    zeros = jnp.zeros((L,), jnp.float32)
    negs = jnp.full((L,), NEG, jnp.float32)
    lane = lax.iota(jnp.int32, L)

    def body_zero(i, _):
        y_v[pl.ds(i * L, L)] = zeros
        return 0

    lax.fori_loop(0, NV, body_zero, 0)

    for r in range(ROWS_PER_W):
        row_id = wid * ROWS_PER_W + r
        pltpu.sync_copy(x_hbm.at[row_id], row_v)

        def body_cm(c, gmax):
            cm = negs
            for j in range(CV):
                cm = jnp.maximum(cm, row_v[pl.ds(c * CH + j * L, L)])
            cm_v[pl.ds(c * L, L)] = cm
            return jnp.maximum(gmax, cm)

        m = jnp.max(lax.fori_loop(0, NCH, body_cm, negs))
        lo0 = m - 0.5

        def body_comp(c, carry):
            k_s, ncmax = carry
            cm = cm_v[pl.ds(c * L, L)]

            def active(car):
                k_a, nc_a = car
                for j in range(CV):
                    v = row_v[pl.ds(c * CH + j * L, L)]
                    msk = v > lo0
                    cum = plsc.cumsum(jnp.where(msk, 1, 0))
                    pos = jnp.full((L,), k_a, jnp.int32) + cum - 1
                    pos = jnp.minimum(pos, CAP + L - 1)
                    plsc.store_scatter(cand_v, [pos], v, mask=msk)
                    idxv = lane + (c * CH + j * L)
                    plsc.store_scatter(cidx_v, [pos], idxv, mask=msk)
                    k_a = k_a + jnp.max(cum)
                    nc_a = jnp.maximum(nc_a, jnp.where(msk, NEG, v))
                return k_a, nc_a

            def inactive(car):
                k_i, nc_i = car
                return k_i, jnp.maximum(nc_i, cm)

            return lax.cond(jnp.any(cm > lo0), active, inactive, (k_s, ncmax))

        k_s, ncmax_v = lax.fori_loop(
            0, NCH, body_comp, (jnp.zeros((), jnp.int32), negs)
        )
        ncmax = jnp.max(ncmax_v)
        kc = jnp.where(k_s < CAP, k_s, CAP)
        cand_v[pl.ds(kc, L)] = negs
        fast = k_s <= CAP
        nvec = lax.shift_right_logical(k_s + (L - 1), 4)

        @pl.when(fast)
        def _fast():
            def bis(_, carry):
                lo, hi = carry
                mid = 0.5 * (lo + hi)

                def body_g(i, acc):
                    return acc + jnp.maximum(cand_v[pl.ds(i * L, L)] - mid, 0.0)

                g = jnp.sum(lax.fori_loop(0, nvec, body_g, zeros))
                pred = g >= 0.5
                return jnp.where(pred, mid, lo), jnp.where(pred, hi, mid)

            t, _ = lax.fori_loop(0, BIS_ITERS, bis, (lo0, m))

            def body_stats(i, carry):
                cntv, sv, bv = carry
                v = cand_v[pl.ds(i * L, L)]
                z = v - m
                msk = v > t
                cntv = cntv + jnp.where(msk, 1.0, 0.0)
                sv = sv + jnp.where(msk, z, 0.0)
                bv = jnp.maximum(bv, jnp.where(msk, NEG, z))
                return cntv, sv, bv

            cntv, sv, bv = lax.fori_loop(
                0, nvec, body_stats, (zeros, zeros, negs)
            )
            cnt = jnp.sum(cntv)
            s_above = jnp.sum(sv)
            v_next = jnp.maximum(jnp.max(bv), ncmax - m)
            num = jnp.full((L,), s_above + v_next - 0.5, jnp.float32)
            den = jnp.full((L,), cnt + 1.0, jnp.float32)
            tau_abs = jnp.full((L,), v_next + m, jnp.float32) - num / den

            def body_y(i, acc):
                z = cand_v[pl.ds(i * L, L)] - tau_abs
                y = jnp.where(z > 0.0, _sqrt16(z), 0.0)
                cand_v[pl.ds(i * L, L)] = y
                return acc + y

            yacc = lax.fori_loop(0, nvec, body_y, zeros)
            inv = jnp.ones((L,), jnp.float32) / jnp.full(
                (L,), jnp.sum(yacc), jnp.float32
            )

            def body_scat(i, _):
                y = cand_v[pl.ds(i * L, L)] * inv
                idxv = cidx_v[pl.ds(i * L, L)]
                msk = (lane + i * L) < k_s
                plsc.store_scatter(y_v, [idxv], y, mask=msk)
                return 0

            lax.fori_loop(0, nvec, body_scat, 0)

        @pl.when(jnp.logical_not(fast))
        def _slow():
            def bis(_, carry):
                lo, hi = carry
                mid = 0.5 * (lo + hi)

                def body_g(i, acc):
                    return acc + jnp.maximum(row_v[pl.ds(i * L, L)] - mid, 0.0)

                g = jnp.sum(lax.fori_loop(0, NV, body_g, zeros))
                pred = g >= 0.5
                return jnp.where(pred, mid, lo), jnp.where(pred, hi, mid)

            t, _ = lax.fori_loop(0, BIS_ITERS, bis, (lo0, m))

            def body_stats(i, carry):
                cntv, sv, bv = carry
                v = row_v[pl.ds(i * L, L)]
                z = v - m
                msk = v > t
                cntv = cntv + jnp.where(msk, 1.0, 0.0)
                sv = sv + jnp.where(msk, z, 0.0)
                bv = jnp.maximum(bv, jnp.where(msk, NEG, z))
                return cntv, sv, bv

            cntv, sv, bv = lax.fori_loop(
                0, NV, body_stats, (zeros, zeros, negs)
            )
            cnt = jnp.sum(cntv)
            s_above = jnp.sum(sv)
            v_next = jnp.max(bv)
            num = jnp.full((L,), s_above + v_next - 0.5, jnp.float32)
            den = jnp.full((L,), cnt + 1.0, jnp.float32)
            tau_abs = jnp.full((L,), v_next + m, jnp.float32) - num / den

            def body_y(i, acc):
                z = row_v[pl.ds(i * L, L)] - tau_abs
                y = jnp.where(z > 0.0, _sqrt16(z), 0.0)
                y_v[pl.ds(i * L, L)] = y
                return acc + y

            yacc = lax.fori_loop(0, NV, body_y, zeros)
            inv = jnp.ones((L,), jnp.float32) / jnp.full(
                (L,), jnp.sum(yacc), jnp.float32
            )

            def body_scale(i, _):
                y_v[pl.ds(i * L, L)] = y_v[pl.ds(i * L, L)] * inv
                return 0

            lax.fori_loop(0, NV, body_scale, 0)

        pltpu.sync_copy(y_v, out_hbm.at[row_id])

        if r < ROWS_PER_W - 1:
            @pl.when(fast)
            def _rz_fast():
                def body_rz(i, _):
                    idxv = cidx_v[pl.ds(i * L, L)]
                    msk = (lane + i * L) < k_s
                    plsc.store_scatter(y_v, [idxv], zeros, mask=msk)
                    return 0

                lax.fori_loop(0, nvec, body_rz, 0)

            @pl.when(jnp.logical_not(fast))
            def _rz_slow():
                def body_rz(i, _):
                    y_v[pl.ds(i * L, L)] = zeros
                    return 0

                lax.fori_loop(0, NV, body_rz, 0)


def kernel(x):
    return _entmax_sc(x)
